# baseline (speedup 1.0000x reference)
import jax
import jax.numpy as jnp
from jax import lax
from jax.experimental import pallas as pl
from jax.experimental.pallas import tpu as pltpu
from jax.experimental.pallas import tpu_sc as plsc

_NB = 4096
_NL = 200
_D = 64
_NW = 32
_BBLK = _NB // _NW
_BPW = _BBLK * _NL
_NPAIR = _NL // 2


def _emb_body(x_hbm, table_hbm, out_hbm, x_v, idxp_v, rows_v, tile_v,
              s_g0, s_g1, s_o0, s_o1):
    s_gat = (s_g0, s_g1)
    s_out = (s_o0, s_o1)
    wid = lax.axis_index("s") * 2 + lax.axis_index("c")
    base = wid * _BPW

    iota = lax.iota(jnp.int32, 16)
    i200 = iota * _NL

    def build_idx(l, s):
        for c in range(8):
            v = plsc.load_gather(x_v, [i200 + (l + c * 16 * _NL)])
            idxp_v[s, pl.ds(c * 16, 16)] = v

    def gat_cp(s):
        return pltpu.make_async_copy(
            table_hbm.at[idxp_v.at[s]], rows_v.at[s], s_gat[s])

    def transpose(s):
        def f_step(f, carry):
            fs = jnp.full((16,), f, jnp.int32)
            for c in range(8):
                v = plsc.load_gather(rows_v.at[s], [iota + c * 16, fs])
                tile_v[s, pl.ds(f * 128 + c * 16, 16)] = v
            return carry
        lax.fori_loop(0, _D, f_step, 0)

    def out_cp(l, s, fg):
        dst = out_hbm.at[pl.ds(l * 262144 + fg * 32768 + wid * 1024, 1024)]
        return pltpu.make_async_copy(
            tile_v.at[s, pl.ds(fg * 1024, 1024)], dst, s_out[s])

    pltpu.sync_copy(x_hbm.at[pl.ds(base, _BPW)], x_v)
    for s in range(2):
        build_idx(s, s)
        gat_cp(s).start()

    def pair(p, carry):
        for s in range(2):
            l = p * 2 + s

            @pl.when(p >= 1)
            def _():
                for fg in range(8):
                    out_cp(l, s, fg).wait()

            gat_cp(s).wait()
            transpose(s)
            for fg in range(8):
                out_cp(l, s, fg).start()

            @pl.when(p <= _NPAIR - 2)
            def _():
                build_idx(l + 2, s)
                gat_cp(s).start()
        return carry

    lax.fori_loop(0, _NPAIR, pair, 0)

    for s in range(2):
        for fg in range(8):
            out_cp(_NL - 2 + s, s, fg).wait()


@jax.jit
def kernel(x, table):
    xf = x.reshape(_NB * _NL)
    mesh = plsc.VectorSubcoreMesh(core_axis_name="c", subcore_axis_name="s")
    p = pl.kernel(
        _emb_body,
        mesh=mesh,
        compiler_params=pltpu.CompilerParams(
            use_tc_tiling_on_sc=False, needs_layout_passes=False),
        out_type=jax.ShapeDtypeStruct((_NB * _NL * _D,), jnp.float32),
        scratch_types=[
            pltpu.VMEM((_BPW,), jnp.int32),
            pltpu.VMEM((2, _BBLK), jnp.int32),
            pltpu.VMEM((2, _BBLK, _D), jnp.float32),
            pltpu.VMEM((2, _BBLK * _D), jnp.float32),
            pltpu.SemaphoreType.DMA,
            pltpu.SemaphoreType.DMA,
            pltpu.SemaphoreType.DMA,
            pltpu.SemaphoreType.DMA,
        ],
    )(xf, table)
    out = (p.reshape(_NL, 8, _NW, 8, 128)
            .transpose(2, 4, 0, 1, 3)
            .reshape(_NB, _NL, _D))
    return out
